# Initial kernel scaffold; baseline (speedup 1.0000x reference)
import jax
import jax.numpy as jnp
from jax.experimental import pallas as pl


def kernel(cond_indices, emb0, emb1, emb2, emb3, emb4, emb5, emb6, W, b):
    raise NotImplementedError("write your pallas kernel here")


# speedup vs baseline: 3.3550x; 3.3550x over previous
import functools

import jax
import jax.numpy as jnp
import numpy as np
from jax import lax
from jax.experimental import pallas as pl
from jax.experimental.pallas import tpu as pltpu
from jax.experimental.pallas import tpu_sc as plsc

_NC, _NS, _L = 2, 16, 16
_NW = _NC * _NS

_NDIM = 7
_E = 64
_D = _NDIM * _E
_NCOMB = 3 ** _NDIM
_FPAD = 2304
_SROWS = 32


@functools.cache
def _combo_onehot():
    c = np.arange(_FPAD)
    a = np.zeros((_FPAD, _SROWS), np.float32)
    for i in range(_NDIM):
        a[c, 3 * i + (c // 3 ** i) % 3] = 1.0
    return jnp.asarray(a)


def _prep_body(a_ref, s_ref, w_ref, b_ref, f_ref):
    p = lax.dot_general(
        s_ref[...], w_ref[...], (((1,), (1,)), ((), ())),
        preferred_element_type=jnp.float32,
    )
    acc = lax.dot_general(
        a_ref[...], p, (((1,), (0,)), ((), ())),
        preferred_element_type=jnp.float32,
    ) + b_ref[...]
    f_ref[...] = jnp.where(acc >= 0.0, acc, 0.2 * acc)


_prep = pl.pallas_call(
    _prep_body,
    out_shape=jax.ShapeDtypeStruct((_FPAD, _D), jnp.float32),
)


@functools.cache
def _radix_w():
    return jnp.asarray([[3 ** i for i in range(_NDIM)]], jnp.int32)


def _combine_body(w_ref, idx_ref, comb_ref):
    comb_ref[...] = jnp.sum(idx_ref[...] * w_ref[...], axis=1).reshape(128, 128)


_combine = pl.pallas_call(
    _combine_body,
    out_shape=jax.ShapeDtypeStruct((128, 128), jnp.int32),
)


@functools.cache
def _make_lookup(batch):
    b_per_w = batch // _NW
    chunk = 128
    nchunk = b_per_w // chunk
    mesh = plsc.VectorSubcoreMesh(core_axis_name="c", subcore_axis_name="s")

    @functools.partial(
        pl.kernel,
        mesh=mesh,
        compiler_params=pltpu.CompilerParams(use_tc_tiling_on_sc=False),
        out_type=jax.ShapeDtypeStruct((batch, _D), jnp.float32),
        scratch_types=[
            pltpu.VMEM((nchunk, chunk), jnp.int32),
            pltpu.VMEM((chunk, _D), jnp.float32),
            pltpu.VMEM((chunk, _D), jnp.float32),
            pltpu.SemaphoreType.DMA,
            pltpu.SemaphoreType.DMA,
        ],
    )
    def lookup(comb_hbm, f_hbm, out_hbm, combv, rows_a, rows_b, gsem, wsem):
        wid = lax.axis_index("s") * _NC + lax.axis_index("c")
        base = wid * b_per_w
        pltpu.sync_copy(comb_hbm.at[pl.ds(wid * nchunk, nchunk)], combv)
        bufs = [rows_a, rows_b]
        writes = [None, None]
        gprev = pltpu.async_copy(f_hbm.at[combv.at[0]], bufs[0], gsem)
        for j in range(nchunk):
            buf = bufs[j % 2]
            gprev.wait()
            if j + 1 < nchunk:
                nxt = bufs[(j + 1) % 2]
                if writes[(j + 1) % 2] is not None:
                    writes[(j + 1) % 2].wait()
                gprev = pltpu.async_copy(f_hbm.at[combv.at[j + 1]], nxt, gsem)
            writes[j % 2] = pltpu.async_copy(
                buf, out_hbm.at[pl.ds(base + j * chunk, chunk)], wsem)
        writes[(nchunk - 2) % 2].wait()
        writes[(nchunk - 1) % 2].wait()

    return lookup


def kernel(cond_indices, emb0, emb1, emb2, emb3, emb4, emb5, emb6, W, b):
    tables = [emb0, emb1, emb2, emb3, emb4, emb5, emb6]
    s = jnp.zeros((_SROWS, _D), jnp.float32)
    for i, t in enumerate(tables):
        s = lax.dynamic_update_slice(s, t[:3, :], (3 * i, _E * i))
    f = _prep(_combo_onehot(), s, W, b.reshape(1, _D))
    comb = _combine(_radix_w(), cond_indices)
    return _make_lookup(cond_indices.shape[0])(comb, f)


# speedup vs baseline: 4.7035x; 1.4020x over previous
import functools

import jax
import jax.numpy as jnp
import numpy as np
from jax import lax
from jax.experimental import pallas as pl
from jax.experimental.pallas import tpu as pltpu
from jax.experimental.pallas import tpu_sc as plsc

_NC, _NS, _L = 2, 16, 16
_NW = _NC * _NS

_NDIM = 7
_E = 64
_D = _NDIM * _E
_DPAD = 512
_NCOMB = 3 ** _NDIM
_FPAD = 2304
_SROWS = 32


@functools.cache
def _combo_onehot():
    c = np.arange(_FPAD)
    a = np.zeros((_FPAD, _SROWS), np.float32)
    for i in range(_NDIM):
        a[c, 3 * i + (c // 3 ** i) % 3] = 1.0
    return jnp.asarray(a)


def _prep_body(a_ref, s_ref, w_ref, b_ref, f_ref):
    p = lax.dot_general(
        s_ref[...], w_ref[...], (((1,), (1,)), ((), ())),
        preferred_element_type=jnp.float32,
    )
    acc = lax.dot_general(
        a_ref[...], p, (((1,), (0,)), ((), ())),
        preferred_element_type=jnp.float32,
    ) + b_ref[...]
    f_ref[:, : _D] = jnp.where(acc >= 0.0, acc, 0.2 * acc)
    f_ref[:, _D:] = jnp.zeros((_FPAD, _DPAD - _D), jnp.float32)


_prep = pl.pallas_call(
    _prep_body,
    out_shape=jax.ShapeDtypeStruct((_FPAD, _DPAD), jnp.float32),
)


@functools.cache
def _radix_w():
    return jnp.asarray([[3 ** i for i in range(_NDIM)]], jnp.int32)


def _combine_body(w_ref, idx_ref, comb_ref):
    comb_ref[...] = jnp.sum(idx_ref[...] * w_ref[...], axis=1).reshape(128, 128)


_combine = pl.pallas_call(
    _combine_body,
    out_shape=jax.ShapeDtypeStruct((128, 128), jnp.int32),
)


@functools.cache
def _make_lookup(batch):
    b_per_w = batch // _NW
    chunk = 64
    nchunk = b_per_w // chunk
    mesh = plsc.VectorSubcoreMesh(core_axis_name="c", subcore_axis_name="s")

    @functools.partial(
        pl.kernel,
        mesh=mesh,
        out_type=jax.ShapeDtypeStruct((batch, _DPAD), jnp.float32),
        scratch_types=[
            pltpu.VMEM((b_per_w,), jnp.int32),
            pltpu.VMEM((chunk, _DPAD), jnp.float32),
            pltpu.VMEM((chunk, _DPAD), jnp.float32),
            pltpu.SemaphoreType.DMA,
            pltpu.SemaphoreType.DMA,
        ],
    )
    def lookup(comb_hbm, f_hbm, out_hbm, combv, rows_a, rows_b, gsem, wsem):
        wid = lax.axis_index("s") * _NC + lax.axis_index("c")
        base = wid * b_per_w
        pltpu.sync_copy(comb_hbm.at[pl.ds(base, b_per_w)], combv)
        bufs = [rows_a, rows_b]
        writes = [None, None]
        gprev = pltpu.async_copy(
            f_hbm.at[combv.at[pl.ds(0, chunk)]], bufs[0], gsem)
        for j in range(nchunk):
            buf = bufs[j % 2]
            gprev.wait()
            if j + 1 < nchunk:
                nxt = bufs[(j + 1) % 2]
                if writes[(j + 1) % 2] is not None:
                    writes[(j + 1) % 2].wait()
                gprev = pltpu.async_copy(
                    f_hbm.at[combv.at[pl.ds((j + 1) * chunk, chunk)]],
                    nxt, gsem)
            writes[j % 2] = pltpu.async_copy(
                buf, out_hbm.at[pl.ds(base + j * chunk, chunk)], wsem)
        writes[(nchunk - 2) % 2].wait()
        writes[(nchunk - 1) % 2].wait()

    return lookup


def kernel(cond_indices, emb0, emb1, emb2, emb3, emb4, emb5, emb6, W, b):
    tables = [emb0, emb1, emb2, emb3, emb4, emb5, emb6]
    s = jnp.zeros((_SROWS, _D), jnp.float32)
    for i, t in enumerate(tables):
        s = lax.dynamic_update_slice(s, t[:3, :], (3 * i, _E * i))
    f = _prep(_combo_onehot(), s, W, b.reshape(1, _D))
    comb = _combine(_radix_w(), cond_indices)
    out = _make_lookup(cond_indices.shape[0])(comb.reshape(-1), f)
    return out[:, :_D]
